# baseline (speedup 1.0000x reference)
import functools

import jax
import jax.numpy as jnp
from jax import lax
from jax.experimental import pallas as pl
from jax.experimental.pallas import tpu as pltpu
from jax.experimental.pallas import tpu_sc as plsc

N = 10000
D = 128
C = 16
E = 320000

NC = 2
NS = 16
NW = NC * NS
B = 64
CH = 160
PH = 4
PCH = CH // PH
NB = 4
EPW = CH * B
E_PAD = NW * EPW
NPAD = 10240
RPS = NPAD // NS
RCH = RPS // B

BK = 1000


def _sc_agg_body(src_hbm, dst_hbm, x_hbm, msum_hbm,
                 src_v, dst_v, rows_v, msum_sh, semg, sems):
    c = lax.axis_index("c")
    s = lax.axis_index("s")

    zero16 = jnp.zeros((16,), jnp.float32)

    def _zrow(i, _):
        for j in range(8):
            rows_v[0, i, pl.ds(j * 16, 16)] = zero16
        return 0
    lax.fori_loop(0, B, _zrow, 0)
    for k in range(RCH):
        pltpu.sync_copy(rows_v.at[0], msum_sh.at[pl.ds(s * RPS + k * B, B)])

    plsc.subcore_barrier()

    w = c * NS + s
    for ph in range(PH):
        pltpu.sync_copy(src_hbm.at[w, pl.ds(ph * PCH, PCH)], src_v)
        pltpu.sync_copy(dst_hbm.at[w, pl.ds(ph * PCH, PCH)], dst_v)
        pltpu.async_copy(x_hbm.at[src_v.at[0]], rows_v.at[0], semg.at[0])
        pltpu.async_copy(x_hbm.at[src_v.at[1]], rows_v.at[1], semg.at[1])

        def _quad(p, _):
            j0 = NB * p
            for r in range(NB):
                j = j0 + r
                r2 = (r + 2) % NB
                pltpu.make_async_copy(
                    x_hbm.at[src_v.at[j]], rows_v.at[r], semg.at[r]).wait()
                pltpu.async_copy(
                    rows_v.at[r], msum_sh.at[dst_v.at[j]], sems.at[r], add=True)

                @pl.when(j >= 2)
                def _ws():
                    pltpu.make_async_copy(
                        rows_v.at[r2], msum_sh.at[dst_v.at[j]],
                        sems.at[r2]).wait()

                @pl.when(j + 2 < PCH)
                def _gs():
                    pltpu.async_copy(
                        x_hbm.at[src_v.at[j + 2]], rows_v.at[r2], semg.at[r2])
            return 0
        lax.fori_loop(0, PCH // NB, _quad, 0)

        pltpu.make_async_copy(
            rows_v.at[(PCH - 2) % NB], msum_sh.at[dst_v.at[PCH - 1]],
            sems.at[(PCH - 2) % NB]).wait()
        pltpu.make_async_copy(
            rows_v.at[(PCH - 1) % NB], msum_sh.at[dst_v.at[PCH - 1]],
            sems.at[(PCH - 1) % NB]).wait()

    plsc.subcore_barrier()

    for k in range(RCH):
        r0 = s * RPS + k * B
        pltpu.sync_copy(msum_sh.at[pl.ds(r0, B)], msum_hbm.at[c, pl.ds(r0, B)])


def _make_agg():
    mesh = plsc.VectorSubcoreMesh(core_axis_name="c", subcore_axis_name="s")
    return pl.kernel(
        _sc_agg_body,
        out_type=jax.ShapeDtypeStruct((NC, NPAD, 128), jnp.float32),
        mesh=mesh,
        scratch_types=[
            pltpu.VMEM((PCH, B), jnp.int32),
            pltpu.VMEM((PCH, B), jnp.int32),
            pltpu.VMEM((NB, B, 128), jnp.float32),
            pltpu.VMEM_SHARED((NPAD, 128), jnp.float32),
            pltpu.SemaphoreType.DMA((NB,)),
            pltpu.SemaphoreType.DMA((NB,)),
        ],
    )


def _sc_cnt_body(dst_hbm, cnt_hbm, dst_v, buf_v, cnt_sh):
    c = lax.axis_index("c")
    s = lax.axis_index("s")

    zero16 = jnp.zeros((16,), jnp.float32)
    one16 = zero16 + 1.0

    def _zrow(i, _):
        for j in range(8):
            buf_v[i, pl.ds(j * 16, 16)] = zero16
        return 0
    lax.fori_loop(0, B, _zrow, 0)
    for k in range(RCH):
        pltpu.sync_copy(buf_v, cnt_sh.at[pl.ds(s * RPS + k * B, B)])

    def _orow(i, _):
        for j in range(8):
            buf_v[i, pl.ds(j * 16, 16)] = one16
        return 0
    lax.fori_loop(0, B, _orow, 0)

    plsc.subcore_barrier()

    pltpu.sync_copy(dst_hbm.at[c * NS + s], dst_v)

    def _cstep(j, _):
        pltpu.sync_copy(buf_v, cnt_sh.at[dst_v.at[j]], add=True)
        return 0
    lax.fori_loop(0, CH, _cstep, 0)

    plsc.subcore_barrier()

    for k in range(RCH):
        r0 = s * RPS + k * B
        pltpu.sync_copy(cnt_sh.at[pl.ds(r0, B)], cnt_hbm.at[c, pl.ds(r0, B)])


def _make_cnt():
    mesh = plsc.VectorSubcoreMesh(core_axis_name="c", subcore_axis_name="s")
    return pl.kernel(
        _sc_cnt_body,
        out_type=jax.ShapeDtypeStruct((NC, NPAD, 128), jnp.float32),
        mesh=mesh,
        scratch_types=[
            pltpu.VMEM((CH, B), jnp.int32),
            pltpu.VMEM((B, 128), jnp.float32),
            pltpu.VMEM_SHARED((NPAD, 128), jnp.float32),
        ],
    )


def _tc_conv_body(msum_ref, cnt_ref, xin_ref, wl_ref, wr_ref, b_ref, out_ref):
    inv = 1.0 / jnp.maximum(cnt_ref[0] + cnt_ref[1], 1.0)
    mean = (msum_ref[0] + msum_ref[1]) * inv
    acc = jnp.dot(mean, wl_ref[...], preferred_element_type=jnp.float32,
                  precision=jax.lax.Precision.HIGHEST)
    acc = acc + jnp.dot(xin_ref[...], wr_ref[...],
                        preferred_element_type=jnp.float32,
                  precision=jax.lax.Precision.HIGHEST)
    out_ref[...] = jnp.maximum(acc + b_ref[...], 0.0)


def _tc_conv(msum, cnt, xin, wl, wr, b):
    return pl.pallas_call(
        _tc_conv_body,
        grid=(N // BK,),
        in_specs=[
            pl.BlockSpec((2, BK, 128), lambda j: (0, j, 0)),
            pl.BlockSpec((2, BK, 128), lambda j: (0, j, 0)),
            pl.BlockSpec((BK, 128), lambda j: (j, 0)),
            pl.BlockSpec((128, 128), lambda j: (0, 0)),
            pl.BlockSpec((128, 128), lambda j: (0, 0)),
            pl.BlockSpec((1, 128), lambda j: (0, 0)),
        ],
        out_specs=pl.BlockSpec((BK, 128), lambda j: (j, 0)),
        out_shape=jax.ShapeDtypeStruct((N, 128), jnp.float32),
    )(msum, cnt, xin, wl, wr, b)


def _tc_final_body(msum_ref, cnt_ref, xin_ref, wl_ref, wr_ref, b_ref,
                   x1_ref, wc1_ref, bc1_ref, wc2_ref, bc2_ref, out_ref):
    inv = 1.0 / jnp.maximum(cnt_ref[0] + cnt_ref[1], 1.0)
    mean = (msum_ref[0] + msum_ref[1]) * inv
    acc = jnp.dot(mean, wl_ref[...], preferred_element_type=jnp.float32,
                  precision=jax.lax.Precision.HIGHEST)
    acc = acc + jnp.dot(xin_ref[...], wr_ref[...],
                        preferred_element_type=jnp.float32,
                  precision=jax.lax.Precision.HIGHEST)
    x3 = jnp.maximum(acc + b_ref[...], 0.0)
    h = x1_ref[...] + x3
    hc = jnp.dot(h, wc1_ref[...], preferred_element_type=jnp.float32,
                  precision=jax.lax.Precision.HIGHEST)
    hc = jnp.maximum(hc + bc1_ref[...], 0.0)
    out_ref[...] = jnp.dot(hc, wc2_ref[...],
                           preferred_element_type=jnp.float32,
                  precision=jax.lax.Precision.HIGHEST) + bc2_ref[...]


def _tc_final(msum, cnt, xin, wl, wr, b, x1, wc1, bc1, wc2, bc2):
    return pl.pallas_call(
        _tc_final_body,
        grid=(N // BK,),
        in_specs=[
            pl.BlockSpec((2, BK, 128), lambda j: (0, j, 0)),
            pl.BlockSpec((2, BK, 128), lambda j: (0, j, 0)),
            pl.BlockSpec((BK, 128), lambda j: (j, 0)),
            pl.BlockSpec((128, 128), lambda j: (0, 0)),
            pl.BlockSpec((128, 128), lambda j: (0, 0)),
            pl.BlockSpec((1, 128), lambda j: (0, 0)),
            pl.BlockSpec((BK, 128), lambda j: (j, 0)),
            pl.BlockSpec((128, 64), lambda j: (0, 0)),
            pl.BlockSpec((1, 64), lambda j: (0, 0)),
            pl.BlockSpec((64, 16), lambda j: (0, 0)),
            pl.BlockSpec((1, 16), lambda j: (0, 0)),
        ],
        out_specs=pl.BlockSpec((BK, 16), lambda j: (j, 0)),
        out_shape=jax.ShapeDtypeStruct((N, C), jnp.float32),
    )(msum, cnt, xin, wl, wr, b, x1, wc1, bc1, wc2, bc2)


def kernel(x, edge_index, Wl11, Wr11, b11, Wl12, Wr12, b12,
           Wl21, Wr21, b21, Wl22, Wr22, b22,
           Wl31, Wr31, b31, Wl32, Wr32, b32,
           Wc1, bc1, Wc2, bc2):
    src = edge_index[0]
    dst = edge_index[1]
    pad = E_PAD - E
    ar = jnp.arange(pad, dtype=jnp.int32)
    src_p = jnp.concatenate([src, (ar * 97) % N]).reshape(NW, CH, B)
    dst_p = jnp.concatenate([dst, N + (ar % (NPAD - N))]).reshape(NW, CH, B)

    agg = _make_agg()
    cnt = _make_cnt()(dst_p)

    msum = agg(src_p, dst_p, x)
    h = _tc_conv(msum, cnt, x, Wl11, Wr11, b11.reshape(1, -1))
    msum = agg(src_p, dst_p, h)
    x1 = _tc_conv(msum, cnt, h, Wl12, Wr12, b12.reshape(1, -1))
    msum = agg(src_p, dst_p, x1)
    h = _tc_conv(msum, cnt, x1, Wl21, Wr21, b21.reshape(1, -1))
    msum = agg(src_p, dst_p, h)
    x2 = _tc_conv(msum, cnt, h, Wl22, Wr22, b22.reshape(1, -1))
    msum = agg(src_p, dst_p, x2)
    h = _tc_conv(msum, cnt, x2, Wl31, Wr31, b31.reshape(1, -1))
    msum = agg(src_p, dst_p, h)
    return _tc_final(msum, cnt, h, Wl32, Wr32, b32.reshape(1, -1),
                     x1, Wc1, bc1.reshape(1, -1), Wc2, bc2.reshape(1, -1))


# speedup vs baseline: 1.1363x; 1.1363x over previous
import functools

import jax
import jax.numpy as jnp
from jax import lax
from jax.experimental import pallas as pl
from jax.experimental.pallas import tpu as pltpu
from jax.experimental.pallas import tpu_sc as plsc

N = 10000
D = 128
C = 16
E = 320000

NC = 2
NS = 16
NW = NC * NS
B = 128
CH = 80
PH = 2
PCH = CH // PH
EPW = CH * B
E_PAD = NW * EPW
NPAD = 10240
RPS = NPAD // NS
RCH = RPS // B

BK = 1000


def _sc_agg_body(src_hbm, dst_hbm, x_hbm, msum_hbm,
                 src_v, dst_v, rows2_v, msum_sh, sem0, sem1):
    c = lax.axis_index("c")
    s = lax.axis_index("s")

    zero16 = jnp.zeros((16,), jnp.float32)

    def _zrow(i, _):
        for j in range(8):
            rows2_v[0, i, pl.ds(j * 16, 16)] = zero16
        return 0
    lax.fori_loop(0, B, _zrow, 0)
    for k in range(RCH):
        pltpu.sync_copy(rows2_v.at[0], msum_sh.at[pl.ds(s * RPS + k * B, B)])

    plsc.subcore_barrier()

    w = c * NS + s
    for ph in range(PH):
        pltpu.sync_copy(src_hbm.at[w, pl.ds(ph * PCH, PCH)], src_v)
        pltpu.sync_copy(dst_hbm.at[w, pl.ds(ph * PCH, PCH)], dst_v)
        pltpu.async_copy(x_hbm.at[src_v.at[0]], rows2_v.at[0], sem0)
        pltpu.async_copy(x_hbm.at[src_v.at[1]], rows2_v.at[1], sem1)

        def _pair(p, _):
            j0 = 2 * p
            pltpu.make_async_copy(
                x_hbm.at[src_v.at[j0]], rows2_v.at[0], sem0).wait()
            pltpu.sync_copy(rows2_v.at[0], msum_sh.at[dst_v.at[j0]], add=True)

            @pl.when(j0 + 2 < PCH)
            def _g0():
                pltpu.async_copy(
                    x_hbm.at[src_v.at[j0 + 2]], rows2_v.at[0], sem0)

            pltpu.make_async_copy(
                x_hbm.at[src_v.at[j0 + 1]], rows2_v.at[1], sem1).wait()
            pltpu.sync_copy(
                rows2_v.at[1], msum_sh.at[dst_v.at[j0 + 1]], add=True)

            @pl.when(j0 + 3 < PCH)
            def _g1():
                pltpu.async_copy(
                    x_hbm.at[src_v.at[j0 + 3]], rows2_v.at[1], sem1)
            return 0
        lax.fori_loop(0, PCH // 2, _pair, 0)

    plsc.subcore_barrier()

    for k in range(RCH):
        r0 = s * RPS + k * B
        pltpu.sync_copy(msum_sh.at[pl.ds(r0, B)], msum_hbm.at[c, pl.ds(r0, B)])


def _make_agg():
    mesh = plsc.VectorSubcoreMesh(core_axis_name="c", subcore_axis_name="s")
    return pl.kernel(
        _sc_agg_body,
        out_type=jax.ShapeDtypeStruct((NC, NPAD, 128), jnp.float32),
        mesh=mesh,
        scratch_types=[
            pltpu.VMEM((PCH, B), jnp.int32),
            pltpu.VMEM((PCH, B), jnp.int32),
            pltpu.VMEM((2, B, 128), jnp.float32),
            pltpu.VMEM_SHARED((NPAD, 128), jnp.float32),
            pltpu.SemaphoreType.DMA,
            pltpu.SemaphoreType.DMA,
        ],
    )


def _sc_cnt_body(dst_hbm, cnt_hbm, dst_v, buf_v, cnt_sh):
    c = lax.axis_index("c")
    s = lax.axis_index("s")

    zero16 = jnp.zeros((16,), jnp.float32)
    one16 = zero16 + 1.0

    def _zrow(i, _):
        for j in range(8):
            buf_v[i, pl.ds(j * 16, 16)] = zero16
        return 0
    lax.fori_loop(0, B, _zrow, 0)
    for k in range(RCH):
        pltpu.sync_copy(buf_v, cnt_sh.at[pl.ds(s * RPS + k * B, B)])

    def _orow(i, _):
        for j in range(8):
            buf_v[i, pl.ds(j * 16, 16)] = one16
        return 0
    lax.fori_loop(0, B, _orow, 0)

    plsc.subcore_barrier()

    pltpu.sync_copy(dst_hbm.at[c * NS + s], dst_v)

    def _cstep(j, _):
        pltpu.sync_copy(buf_v, cnt_sh.at[dst_v.at[j]], add=True)
        return 0
    lax.fori_loop(0, CH, _cstep, 0)

    plsc.subcore_barrier()

    for k in range(RCH):
        r0 = s * RPS + k * B
        pltpu.sync_copy(cnt_sh.at[pl.ds(r0, B)], cnt_hbm.at[c, pl.ds(r0, B)])


def _make_cnt():
    mesh = plsc.VectorSubcoreMesh(core_axis_name="c", subcore_axis_name="s")
    return pl.kernel(
        _sc_cnt_body,
        out_type=jax.ShapeDtypeStruct((NC, NPAD, 128), jnp.float32),
        mesh=mesh,
        scratch_types=[
            pltpu.VMEM((CH, B), jnp.int32),
            pltpu.VMEM((B, 128), jnp.float32),
            pltpu.VMEM_SHARED((NPAD, 128), jnp.float32),
        ],
    )


def _tc_conv_body(msum_ref, cnt_ref, xin_ref, wl_ref, wr_ref, b_ref, out_ref):
    inv = 1.0 / jnp.maximum(cnt_ref[0] + cnt_ref[1], 1.0)
    mean = (msum_ref[0] + msum_ref[1]) * inv
    acc = jnp.dot(mean, wl_ref[...], preferred_element_type=jnp.float32,
                  precision=jax.lax.Precision.HIGHEST)
    acc = acc + jnp.dot(xin_ref[...], wr_ref[...],
                        preferred_element_type=jnp.float32,
                  precision=jax.lax.Precision.HIGHEST)
    out_ref[...] = jnp.maximum(acc + b_ref[...], 0.0)


def _tc_conv(msum, cnt, xin, wl, wr, b):
    return pl.pallas_call(
        _tc_conv_body,
        grid=(N // BK,),
        in_specs=[
            pl.BlockSpec((2, BK, 128), lambda j: (0, j, 0)),
            pl.BlockSpec((2, BK, 128), lambda j: (0, j, 0)),
            pl.BlockSpec((BK, 128), lambda j: (j, 0)),
            pl.BlockSpec((128, 128), lambda j: (0, 0)),
            pl.BlockSpec((128, 128), lambda j: (0, 0)),
            pl.BlockSpec((1, 128), lambda j: (0, 0)),
        ],
        out_specs=pl.BlockSpec((BK, 128), lambda j: (j, 0)),
        out_shape=jax.ShapeDtypeStruct((N, 128), jnp.float32),
    )(msum, cnt, xin, wl, wr, b)


def _tc_final_body(msum_ref, cnt_ref, xin_ref, wl_ref, wr_ref, b_ref,
                   x1_ref, wc1_ref, bc1_ref, wc2_ref, bc2_ref, out_ref):
    inv = 1.0 / jnp.maximum(cnt_ref[0] + cnt_ref[1], 1.0)
    mean = (msum_ref[0] + msum_ref[1]) * inv
    acc = jnp.dot(mean, wl_ref[...], preferred_element_type=jnp.float32,
                  precision=jax.lax.Precision.HIGHEST)
    acc = acc + jnp.dot(xin_ref[...], wr_ref[...],
                        preferred_element_type=jnp.float32,
                  precision=jax.lax.Precision.HIGHEST)
    x3 = jnp.maximum(acc + b_ref[...], 0.0)
    h = x1_ref[...] + x3
    hc = jnp.dot(h, wc1_ref[...], preferred_element_type=jnp.float32,
                  precision=jax.lax.Precision.HIGHEST)
    hc = jnp.maximum(hc + bc1_ref[...], 0.0)
    out_ref[...] = jnp.dot(hc, wc2_ref[...],
                           preferred_element_type=jnp.float32,
                  precision=jax.lax.Precision.HIGHEST) + bc2_ref[...]


def _tc_final(msum, cnt, xin, wl, wr, b, x1, wc1, bc1, wc2, bc2):
    return pl.pallas_call(
        _tc_final_body,
        grid=(N // BK,),
        in_specs=[
            pl.BlockSpec((2, BK, 128), lambda j: (0, j, 0)),
            pl.BlockSpec((2, BK, 128), lambda j: (0, j, 0)),
            pl.BlockSpec((BK, 128), lambda j: (j, 0)),
            pl.BlockSpec((128, 128), lambda j: (0, 0)),
            pl.BlockSpec((128, 128), lambda j: (0, 0)),
            pl.BlockSpec((1, 128), lambda j: (0, 0)),
            pl.BlockSpec((BK, 128), lambda j: (j, 0)),
            pl.BlockSpec((128, 64), lambda j: (0, 0)),
            pl.BlockSpec((1, 64), lambda j: (0, 0)),
            pl.BlockSpec((64, 16), lambda j: (0, 0)),
            pl.BlockSpec((1, 16), lambda j: (0, 0)),
        ],
        out_specs=pl.BlockSpec((BK, 16), lambda j: (j, 0)),
        out_shape=jax.ShapeDtypeStruct((N, C), jnp.float32),
    )(msum, cnt, xin, wl, wr, b, x1, wc1, bc1, wc2, bc2)


def kernel(x, edge_index, Wl11, Wr11, b11, Wl12, Wr12, b12,
           Wl21, Wr21, b21, Wl22, Wr22, b22,
           Wl31, Wr31, b31, Wl32, Wr32, b32,
           Wc1, bc1, Wc2, bc2):
    src = edge_index[0]
    dst = edge_index[1]
    pad = E_PAD - E
    ar = jnp.arange(pad, dtype=jnp.int32)
    src_p = jnp.concatenate([src, (ar * 97) % N]).reshape(NW, CH, B)
    dst_p = jnp.concatenate([dst, N + (ar % (NPAD - N))]).reshape(NW, CH, B)

    agg = _make_agg()
    cnt = _make_cnt()(dst_p)

    msum = agg(src_p, dst_p, x)
    h = _tc_conv(msum, cnt, x, Wl11, Wr11, b11.reshape(1, -1))
    msum = agg(src_p, dst_p, h)
    x1 = _tc_conv(msum, cnt, h, Wl12, Wr12, b12.reshape(1, -1))
    msum = agg(src_p, dst_p, x1)
    h = _tc_conv(msum, cnt, x1, Wl21, Wr21, b21.reshape(1, -1))
    msum = agg(src_p, dst_p, h)
    x2 = _tc_conv(msum, cnt, h, Wl22, Wr22, b22.reshape(1, -1))
    msum = agg(src_p, dst_p, x2)
    h = _tc_conv(msum, cnt, x2, Wl31, Wr31, b31.reshape(1, -1))
    msum = agg(src_p, dst_p, h)
    return _tc_final(msum, cnt, h, Wl32, Wr32, b32.reshape(1, -1),
                     x1, Wc1, bc1.reshape(1, -1), Wc2, bc2.reshape(1, -1))


# speedup vs baseline: 1.1448x; 1.0075x over previous
import functools

import jax
import jax.numpy as jnp
from jax import lax
from jax.experimental import pallas as pl
from jax.experimental.pallas import tpu as pltpu
from jax.experimental.pallas import tpu_sc as plsc

N = 10000
D = 128
C = 16
E = 320000

NC = 2
NS = 16
NW = NC * NS
B = 128
CH = 80
PH = 2
PCH = CH // PH
EPW = CH * B
E_PAD = NW * EPW
NPAD = 10240
RPS = NPAD // NS
RCH = RPS // B
ZR = 32

BK = 1000


def _sc_agg_body(src_hbm, dst_hbm, x_hbm, msum_hbm,
                 src_v, dst_v, rows2_v, zbuf_v, msum_sh, sem0, sem1, semz):
    c = lax.axis_index("c")
    s = lax.axis_index("s")

    zero16 = jnp.zeros((16,), jnp.float32)
    w = c * NS + s

    pltpu.sync_copy(src_hbm.at[w, pl.ds(0, PCH)], src_v)
    pltpu.sync_copy(dst_hbm.at[w, pl.ds(0, PCH)], dst_v)
    pltpu.async_copy(x_hbm.at[src_v.at[0]], rows2_v.at[0], sem0)
    pltpu.async_copy(x_hbm.at[src_v.at[1]], rows2_v.at[1], sem1)

    def _zrow(i, _):
        for j in range(8):
            zbuf_v[i, pl.ds(j * 16, 16)] = zero16
        return 0
    lax.fori_loop(0, ZR, _zrow, 0)
    for k in range(RPS // ZR):
        pltpu.async_copy(
            zbuf_v, msum_sh.at[pl.ds(s * RPS + k * ZR, ZR)], semz)
    for k in range(RPS // ZR):
        pltpu.make_async_copy(
            zbuf_v, msum_sh.at[pl.ds(s * RPS + k * ZR, ZR)], semz).wait()

    plsc.subcore_barrier()

    for ph in range(PH):
        if ph:
            pltpu.sync_copy(src_hbm.at[w, pl.ds(ph * PCH, PCH)], src_v)
            pltpu.sync_copy(dst_hbm.at[w, pl.ds(ph * PCH, PCH)], dst_v)
            pltpu.async_copy(x_hbm.at[src_v.at[0]], rows2_v.at[0], sem0)
            pltpu.async_copy(x_hbm.at[src_v.at[1]], rows2_v.at[1], sem1)

        def _pair(p, _):
            j0 = 2 * p
            pltpu.make_async_copy(
                x_hbm.at[src_v.at[j0]], rows2_v.at[0], sem0).wait()
            pltpu.sync_copy(rows2_v.at[0], msum_sh.at[dst_v.at[j0]], add=True)

            @pl.when(j0 + 2 < PCH)
            def _g0():
                pltpu.async_copy(
                    x_hbm.at[src_v.at[j0 + 2]], rows2_v.at[0], sem0)

            pltpu.make_async_copy(
                x_hbm.at[src_v.at[j0 + 1]], rows2_v.at[1], sem1).wait()
            pltpu.sync_copy(
                rows2_v.at[1], msum_sh.at[dst_v.at[j0 + 1]], add=True)

            @pl.when(j0 + 3 < PCH)
            def _g1():
                pltpu.async_copy(
                    x_hbm.at[src_v.at[j0 + 3]], rows2_v.at[1], sem1)
            return 0
        lax.fori_loop(0, PCH // 2, _pair, 0)

    plsc.subcore_barrier()

    for k in range(RCH):
        r0 = s * RPS + k * B
        pltpu.async_copy(
            msum_sh.at[pl.ds(r0, B)], msum_hbm.at[c, pl.ds(r0, B)], semz)
    for k in range(RCH):
        r0 = s * RPS + k * B
        pltpu.make_async_copy(
            msum_sh.at[pl.ds(r0, B)], msum_hbm.at[c, pl.ds(r0, B)],
            semz).wait()


def _make_agg():
    mesh = plsc.VectorSubcoreMesh(core_axis_name="c", subcore_axis_name="s")
    return pl.kernel(
        _sc_agg_body,
        out_type=jax.ShapeDtypeStruct((NC, NPAD, 128), jnp.float32),
        mesh=mesh,
        scratch_types=[
            pltpu.VMEM((PCH, B), jnp.int32),
            pltpu.VMEM((PCH, B), jnp.int32),
            pltpu.VMEM((2, B, 128), jnp.float32),
            pltpu.VMEM((ZR, 128), jnp.float32),
            pltpu.VMEM_SHARED((NPAD, 128), jnp.float32),
            pltpu.SemaphoreType.DMA,
            pltpu.SemaphoreType.DMA,
            pltpu.SemaphoreType.DMA,
        ],
    )


def _sc_cnt_body(dst_hbm, cnt_hbm, dst_v, buf_v, cnt_sh):
    c = lax.axis_index("c")
    s = lax.axis_index("s")

    zero16 = jnp.zeros((16,), jnp.float32)
    one16 = zero16 + 1.0

    def _zrow(i, _):
        for j in range(8):
            buf_v[i, pl.ds(j * 16, 16)] = zero16
        return 0
    lax.fori_loop(0, B, _zrow, 0)
    for k in range(RCH):
        pltpu.sync_copy(buf_v, cnt_sh.at[pl.ds(s * RPS + k * B, B)])

    def _orow(i, _):
        for j in range(8):
            buf_v[i, pl.ds(j * 16, 16)] = one16
        return 0
    lax.fori_loop(0, B, _orow, 0)

    plsc.subcore_barrier()

    pltpu.sync_copy(dst_hbm.at[c * NS + s], dst_v)

    def _cstep(j, _):
        pltpu.sync_copy(buf_v, cnt_sh.at[dst_v.at[j]], add=True)
        return 0
    lax.fori_loop(0, CH, _cstep, 0)

    plsc.subcore_barrier()

    for k in range(RCH):
        r0 = s * RPS + k * B
        pltpu.sync_copy(cnt_sh.at[pl.ds(r0, B)], cnt_hbm.at[c, pl.ds(r0, B)])


def _make_cnt():
    mesh = plsc.VectorSubcoreMesh(core_axis_name="c", subcore_axis_name="s")
    return pl.kernel(
        _sc_cnt_body,
        out_type=jax.ShapeDtypeStruct((NC, NPAD, 128), jnp.float32),
        mesh=mesh,
        scratch_types=[
            pltpu.VMEM((CH, B), jnp.int32),
            pltpu.VMEM((B, 128), jnp.float32),
            pltpu.VMEM_SHARED((NPAD, 128), jnp.float32),
        ],
    )


def _tc_conv_body(msum_ref, cnt_ref, xin_ref, wl_ref, wr_ref, b_ref, out_ref):
    inv = 1.0 / jnp.maximum(cnt_ref[0] + cnt_ref[1], 1.0)
    mean = (msum_ref[0] + msum_ref[1]) * inv
    acc = jnp.dot(mean, wl_ref[...], preferred_element_type=jnp.float32,
                  precision=jax.lax.Precision.HIGHEST)
    acc = acc + jnp.dot(xin_ref[...], wr_ref[...],
                        preferred_element_type=jnp.float32,
                  precision=jax.lax.Precision.HIGHEST)
    out_ref[...] = jnp.maximum(acc + b_ref[...], 0.0)


def _tc_conv(msum, cnt, xin, wl, wr, b):
    return pl.pallas_call(
        _tc_conv_body,
        grid=(N // BK,),
        in_specs=[
            pl.BlockSpec((2, BK, 128), lambda j: (0, j, 0)),
            pl.BlockSpec((2, BK, 128), lambda j: (0, j, 0)),
            pl.BlockSpec((BK, 128), lambda j: (j, 0)),
            pl.BlockSpec((128, 128), lambda j: (0, 0)),
            pl.BlockSpec((128, 128), lambda j: (0, 0)),
            pl.BlockSpec((1, 128), lambda j: (0, 0)),
        ],
        out_specs=pl.BlockSpec((BK, 128), lambda j: (j, 0)),
        out_shape=jax.ShapeDtypeStruct((N, 128), jnp.float32),
    )(msum, cnt, xin, wl, wr, b)


def _tc_final_body(msum_ref, cnt_ref, xin_ref, wl_ref, wr_ref, b_ref,
                   x1_ref, wc1_ref, bc1_ref, wc2_ref, bc2_ref, out_ref):
    inv = 1.0 / jnp.maximum(cnt_ref[0] + cnt_ref[1], 1.0)
    mean = (msum_ref[0] + msum_ref[1]) * inv
    acc = jnp.dot(mean, wl_ref[...], preferred_element_type=jnp.float32,
                  precision=jax.lax.Precision.HIGHEST)
    acc = acc + jnp.dot(xin_ref[...], wr_ref[...],
                        preferred_element_type=jnp.float32,
                  precision=jax.lax.Precision.HIGHEST)
    x3 = jnp.maximum(acc + b_ref[...], 0.0)
    h = x1_ref[...] + x3
    hc = jnp.dot(h, wc1_ref[...], preferred_element_type=jnp.float32,
                  precision=jax.lax.Precision.HIGHEST)
    hc = jnp.maximum(hc + bc1_ref[...], 0.0)
    out_ref[...] = jnp.dot(hc, wc2_ref[...],
                           preferred_element_type=jnp.float32,
                  precision=jax.lax.Precision.HIGHEST) + bc2_ref[...]


def _tc_final(msum, cnt, xin, wl, wr, b, x1, wc1, bc1, wc2, bc2):
    return pl.pallas_call(
        _tc_final_body,
        grid=(N // BK,),
        in_specs=[
            pl.BlockSpec((2, BK, 128), lambda j: (0, j, 0)),
            pl.BlockSpec((2, BK, 128), lambda j: (0, j, 0)),
            pl.BlockSpec((BK, 128), lambda j: (j, 0)),
            pl.BlockSpec((128, 128), lambda j: (0, 0)),
            pl.BlockSpec((128, 128), lambda j: (0, 0)),
            pl.BlockSpec((1, 128), lambda j: (0, 0)),
            pl.BlockSpec((BK, 128), lambda j: (j, 0)),
            pl.BlockSpec((128, 64), lambda j: (0, 0)),
            pl.BlockSpec((1, 64), lambda j: (0, 0)),
            pl.BlockSpec((64, 16), lambda j: (0, 0)),
            pl.BlockSpec((1, 16), lambda j: (0, 0)),
        ],
        out_specs=pl.BlockSpec((BK, 16), lambda j: (j, 0)),
        out_shape=jax.ShapeDtypeStruct((N, C), jnp.float32),
    )(msum, cnt, xin, wl, wr, b, x1, wc1, bc1, wc2, bc2)


def kernel(x, edge_index, Wl11, Wr11, b11, Wl12, Wr12, b12,
           Wl21, Wr21, b21, Wl22, Wr22, b22,
           Wl31, Wr31, b31, Wl32, Wr32, b32,
           Wc1, bc1, Wc2, bc2):
    src = edge_index[0]
    dst = edge_index[1]
    pad = E_PAD - E
    ar = jnp.arange(pad, dtype=jnp.int32)
    src_p = jnp.concatenate([src, (ar * 97) % N]).reshape(NW, CH, B)
    dst_p = jnp.concatenate([dst, N + (ar % (NPAD - N))]).reshape(NW, CH, B)

    agg = _make_agg()
    cnt = _make_cnt()(dst_p)

    msum = agg(src_p, dst_p, x)
    h = _tc_conv(msum, cnt, x, Wl11, Wr11, b11.reshape(1, -1))
    msum = agg(src_p, dst_p, h)
    x1 = _tc_conv(msum, cnt, h, Wl12, Wr12, b12.reshape(1, -1))
    msum = agg(src_p, dst_p, x1)
    h = _tc_conv(msum, cnt, x1, Wl21, Wr21, b21.reshape(1, -1))
    msum = agg(src_p, dst_p, h)
    x2 = _tc_conv(msum, cnt, h, Wl22, Wr22, b22.reshape(1, -1))
    msum = agg(src_p, dst_p, x2)
    h = _tc_conv(msum, cnt, x2, Wl31, Wr31, b31.reshape(1, -1))
    msum = agg(src_p, dst_p, h)
    return _tc_final(msum, cnt, h, Wl32, Wr32, b32.reshape(1, -1),
                     x1, Wc1, bc1.reshape(1, -1), Wc2, bc2.reshape(1, -1))


# speedup vs baseline: 1.2071x; 1.0544x over previous
import jax
import jax.numpy as jnp
from jax import lax
from jax.experimental import pallas as pl
from jax.experimental.pallas import tpu as pltpu
from jax.experimental.pallas import tpu_sc as plsc

N = 10000
D = 128
C = 16
E = 320000

NC = 2
NS = 16
NW = NC * NS
B = 128
CH = 80
PH = 2
PCH = CH // PH
EPW = CH * B
E_PAD = NW * EPW
NPAD = 10240
RPS = NPAD // NS
RCH = RPS // B
ZR = 32

BK = 2000


def _sc_agg_body(src_hbm, dst_hbm, x_hbm, msum_hbm,
                 src_v, dst_v, rows2_v, zbuf_v, msum_sh, sem0, sem1, semz):
    c = lax.axis_index("c")
    s = lax.axis_index("s")

    zero16 = jnp.zeros((16,), jnp.float32)
    w = c * NS + s

    pltpu.sync_copy(src_hbm.at[w, pl.ds(0, PCH)], src_v)
    pltpu.sync_copy(dst_hbm.at[w, pl.ds(0, PCH)], dst_v)
    pltpu.async_copy(x_hbm.at[src_v.at[0]], rows2_v.at[0], sem0)
    pltpu.async_copy(x_hbm.at[src_v.at[1]], rows2_v.at[1], sem1)

    def _zrow(i, _):
        for j in range(8):
            zbuf_v[i, pl.ds(j * 16, 16)] = zero16
        return 0
    lax.fori_loop(0, ZR, _zrow, 0)
    for k in range(RPS // ZR):
        pltpu.async_copy(
            zbuf_v, msum_sh.at[pl.ds(s * RPS + k * ZR, ZR)], semz)
    for k in range(RPS // ZR):
        pltpu.make_async_copy(
            zbuf_v, msum_sh.at[pl.ds(s * RPS + k * ZR, ZR)], semz).wait()

    plsc.subcore_barrier()

    for ph in range(PH):
        if ph:
            pltpu.sync_copy(src_hbm.at[w, pl.ds(ph * PCH, PCH)], src_v)
            pltpu.sync_copy(dst_hbm.at[w, pl.ds(ph * PCH, PCH)], dst_v)
            pltpu.async_copy(x_hbm.at[src_v.at[0]], rows2_v.at[0], sem0)
            pltpu.async_copy(x_hbm.at[src_v.at[1]], rows2_v.at[1], sem1)

        def _pair(p, _):
            j0 = 2 * p
            pltpu.make_async_copy(
                x_hbm.at[src_v.at[j0]], rows2_v.at[0], sem0).wait()
            pltpu.sync_copy(rows2_v.at[0], msum_sh.at[dst_v.at[j0]], add=True)

            @pl.when(j0 + 2 < PCH)
            def _g0():
                pltpu.async_copy(
                    x_hbm.at[src_v.at[j0 + 2]], rows2_v.at[0], sem0)

            pltpu.make_async_copy(
                x_hbm.at[src_v.at[j0 + 1]], rows2_v.at[1], sem1).wait()
            pltpu.sync_copy(
                rows2_v.at[1], msum_sh.at[dst_v.at[j0 + 1]], add=True)

            @pl.when(j0 + 3 < PCH)
            def _g1():
                pltpu.async_copy(
                    x_hbm.at[src_v.at[j0 + 3]], rows2_v.at[1], sem1)
            return 0
        lax.fori_loop(0, PCH // 2, _pair, 0)

    plsc.subcore_barrier()

    for k in range(RCH):
        r0 = s * RPS + k * B
        pltpu.async_copy(
            msum_sh.at[pl.ds(r0, B)], msum_hbm.at[c, pl.ds(r0, B)], semz)
    for k in range(RCH):
        r0 = s * RPS + k * B
        pltpu.make_async_copy(
            msum_sh.at[pl.ds(r0, B)], msum_hbm.at[c, pl.ds(r0, B)],
            semz).wait()


def _make_agg():
    mesh = plsc.VectorSubcoreMesh(core_axis_name="c", subcore_axis_name="s")
    return pl.kernel(
        _sc_agg_body,
        out_type=jax.ShapeDtypeStruct((NC, NPAD, 128), jnp.float32),
        mesh=mesh,
        scratch_types=[
            pltpu.VMEM((PCH, B), jnp.int32),
            pltpu.VMEM((PCH, B), jnp.int32),
            pltpu.VMEM((2, B, 128), jnp.float32),
            pltpu.VMEM((ZR, 128), jnp.float32),
            pltpu.VMEM_SHARED((NPAD, 128), jnp.float32),
            pltpu.SemaphoreType.DMA,
            pltpu.SemaphoreType.DMA,
            pltpu.SemaphoreType.DMA,
        ],
    )


def _sc_cnt_body(dst_hbm, cnt_hbm, dst_v, buf_v, cnt_sh):
    c = lax.axis_index("c")
    s = lax.axis_index("s")

    zero16 = jnp.zeros((16,), jnp.float32)
    one16 = zero16 + 1.0

    def _zrow(i, _):
        for j in range(8):
            buf_v[i, pl.ds(j * 16, 16)] = zero16
        return 0
    lax.fori_loop(0, B, _zrow, 0)
    for k in range(RCH):
        pltpu.sync_copy(buf_v, cnt_sh.at[pl.ds(s * RPS + k * B, B)])

    def _orow(i, _):
        for j in range(8):
            buf_v[i, pl.ds(j * 16, 16)] = one16
        return 0
    lax.fori_loop(0, B, _orow, 0)

    plsc.subcore_barrier()

    pltpu.sync_copy(dst_hbm.at[c * NS + s], dst_v)

    def _cstep(j, _):
        pltpu.sync_copy(buf_v, cnt_sh.at[dst_v.at[j]], add=True)
        return 0
    lax.fori_loop(0, CH, _cstep, 0)

    plsc.subcore_barrier()

    for k in range(RCH):
        r0 = s * RPS + k * B
        pltpu.sync_copy(cnt_sh.at[pl.ds(r0, B)], cnt_hbm.at[c, pl.ds(r0, B)])


def _make_cnt():
    mesh = plsc.VectorSubcoreMesh(core_axis_name="c", subcore_axis_name="s")
    return pl.kernel(
        _sc_cnt_body,
        out_type=jax.ShapeDtypeStruct((NC, NPAD, 128), jnp.float32),
        mesh=mesh,
        scratch_types=[
            pltpu.VMEM((CH, B), jnp.int32),
            pltpu.VMEM((B, 128), jnp.float32),
            pltpu.VMEM_SHARED((NPAD, 128), jnp.float32),
        ],
    )


def _tc_conv_body(msum_ref, cnt_ref, xin_ref, wl_ref, wr_ref, b_ref, out_ref):
    inv = 1.0 / jnp.maximum(cnt_ref[0] + cnt_ref[1], 1.0)
    mean = (msum_ref[0] + msum_ref[1]) * inv
    acc = jnp.dot(mean, wl_ref[...], preferred_element_type=jnp.float32,
                  precision=jax.lax.Precision.HIGHEST)
    acc = acc + jnp.dot(xin_ref[...], wr_ref[...],
                        preferred_element_type=jnp.float32,
                  precision=jax.lax.Precision.HIGHEST)
    out_ref[...] = jnp.maximum(acc + b_ref[...], 0.0)


def _tc_conv(msum, cnt, xin, wl, wr, b):
    return pl.pallas_call(
        _tc_conv_body,
        grid=(N // BK,),
        in_specs=[
            pl.BlockSpec((2, BK, 128), lambda j: (0, j, 0)),
            pl.BlockSpec((2, BK, 128), lambda j: (0, j, 0)),
            pl.BlockSpec((BK, 128), lambda j: (j, 0)),
            pl.BlockSpec((128, 128), lambda j: (0, 0)),
            pl.BlockSpec((128, 128), lambda j: (0, 0)),
            pl.BlockSpec((1, 128), lambda j: (0, 0)),
        ],
        out_specs=pl.BlockSpec((BK, 128), lambda j: (j, 0)),
        out_shape=jax.ShapeDtypeStruct((N, 128), jnp.float32),
    )(msum, cnt, xin, wl, wr, b)


def _tc_final_body(msum_ref, cnt_ref, xin_ref, wl_ref, wr_ref, b_ref,
                   x1_ref, wc1_ref, bc1_ref, wc2_ref, bc2_ref, out_ref):
    inv = 1.0 / jnp.maximum(cnt_ref[0] + cnt_ref[1], 1.0)
    mean = (msum_ref[0] + msum_ref[1]) * inv
    acc = jnp.dot(mean, wl_ref[...], preferred_element_type=jnp.float32,
                  precision=jax.lax.Precision.HIGHEST)
    acc = acc + jnp.dot(xin_ref[...], wr_ref[...],
                        preferred_element_type=jnp.float32,
                  precision=jax.lax.Precision.HIGHEST)
    x3 = jnp.maximum(acc + b_ref[...], 0.0)
    h = x1_ref[...] + x3
    hc = jnp.dot(h, wc1_ref[...], preferred_element_type=jnp.float32,
                  precision=jax.lax.Precision.HIGHEST)
    hc = jnp.maximum(hc + bc1_ref[...], 0.0)
    out_ref[...] = jnp.dot(hc, wc2_ref[...],
                           preferred_element_type=jnp.float32,
                  precision=jax.lax.Precision.HIGHEST) + bc2_ref[...]


def _tc_final(msum, cnt, xin, wl, wr, b, x1, wc1, bc1, wc2, bc2):
    return pl.pallas_call(
        _tc_final_body,
        grid=(N // BK,),
        in_specs=[
            pl.BlockSpec((2, BK, 128), lambda j: (0, j, 0)),
            pl.BlockSpec((2, BK, 128), lambda j: (0, j, 0)),
            pl.BlockSpec((BK, 128), lambda j: (j, 0)),
            pl.BlockSpec((128, 128), lambda j: (0, 0)),
            pl.BlockSpec((128, 128), lambda j: (0, 0)),
            pl.BlockSpec((1, 128), lambda j: (0, 0)),
            pl.BlockSpec((BK, 128), lambda j: (j, 0)),
            pl.BlockSpec((128, 64), lambda j: (0, 0)),
            pl.BlockSpec((1, 64), lambda j: (0, 0)),
            pl.BlockSpec((64, 16), lambda j: (0, 0)),
            pl.BlockSpec((1, 16), lambda j: (0, 0)),
        ],
        out_specs=pl.BlockSpec((BK, 16), lambda j: (j, 0)),
        out_shape=jax.ShapeDtypeStruct((N, C), jnp.float32),
    )(msum, cnt, xin, wl, wr, b, x1, wc1, bc1, wc2, bc2)


def kernel(x, edge_index, Wl11, Wr11, b11, Wl12, Wr12, b12,
           Wl21, Wr21, b21, Wl22, Wr22, b22,
           Wl31, Wr31, b31, Wl32, Wr32, b32,
           Wc1, bc1, Wc2, bc2):
    src = edge_index[0]
    dst = edge_index[1]
    pad = E_PAD - E
    ar = jnp.arange(pad, dtype=jnp.int32)
    src_p = jnp.concatenate([src, (ar * 97) % N]).reshape(NW, CH, B)
    dst_p = jnp.concatenate([dst, N + (ar % (NPAD - N))]).reshape(NW, CH, B)

    agg = _make_agg()
    cnt = _make_cnt()(dst_p)

    msum = agg(src_p, dst_p, x)
    h = _tc_conv(msum, cnt, x, Wl11, Wr11, b11.reshape(1, -1))
    msum = agg(src_p, dst_p, h)
    x1 = _tc_conv(msum, cnt, h, Wl12, Wr12, b12.reshape(1, -1))
    msum = agg(src_p, dst_p, x1)
    h = _tc_conv(msum, cnt, x1, Wl21, Wr21, b21.reshape(1, -1))
    msum = agg(src_p, dst_p, h)
    x2 = _tc_conv(msum, cnt, h, Wl22, Wr22, b22.reshape(1, -1))
    msum = agg(src_p, dst_p, x2)
    h = _tc_conv(msum, cnt, x2, Wl31, Wr31, b31.reshape(1, -1))
    msum = agg(src_p, dst_p, h)
    return _tc_final(msum, cnt, h, Wl32, Wr32, b32.reshape(1, -1),
                     x1, Wc1, bc1.reshape(1, -1), Wc2, bc2.reshape(1, -1))
